# baseline (device time: 31201 ns/iter reference)
import jax
import jax.numpy as jnp
from jax import lax
from jax.experimental import pallas as pl
from jax.experimental.pallas import tpu as pltpu

N_DEV = 4
DH = 64
B = 2
SQ = 256
D = 768


def _fused(xb, Wq, Wo, K, V):
    Hq = K.shape[1]

    def body(x_ref, wq_ref, wo_ref, k_ref, v_ref, out_ref,
             cA, cB, o_buf, sA, rA, sB, rB):
        my = lax.axis_index("i")
        p1 = my ^ 1
        p2 = 3 - my

        barrier_sem = pltpu.get_barrier_semaphore()
        for nbr in (p1, p2):
            pl.semaphore_signal(
                barrier_sem, inc=1,
                device_id=(nbr,), device_id_type=pl.DeviceIdType.MESH,
            )
        pl.semaphore_wait(barrier_sem, 2)

        def compute_partial(b):
            q = jnp.dot(
                x_ref[b], wq_ref[...], preferred_element_type=jnp.float32
            ).astype(jnp.bfloat16)
            for h in range(Hq):
                qh = q[:, DH * h:DH * (h + 1)]
                s = lax.dot_general(
                    qh, k_ref[b, h],
                    (((1,), (1,)), ((), ())),
                    preferred_element_type=jnp.float32,
                ) * 0.125
                m = jnp.max(s, axis=1, keepdims=True)
                p = jnp.exp(s - m)
                l = jnp.sum(p, axis=1, keepdims=True)
                oh = jnp.dot(
                    p.astype(jnp.bfloat16), v_ref[b, h],
                    preferred_element_type=jnp.float32,
                )
                o_buf[:, DH * h:DH * (h + 1)] = (oh / l).astype(jnp.bfloat16)
            return jnp.dot(
                o_buf[...], wo_ref[...], preferred_element_type=jnp.float32
            )

        def exchange(c, slot_src, slot_dst, sem_s, sem_r, idx, tgt):
            return pltpu.make_async_remote_copy(
                src_ref=c.at[slot_src], dst_ref=c.at[slot_dst],
                send_sem=sem_s.at[idx], recv_sem=sem_r.at[idx],
                device_id=(tgt,), device_id_type=pl.DeviceIdType.MESH,
            )

        pA = compute_partial(0)
        cA[0] = pA.astype(jnp.bfloat16)
        a1 = exchange(cA, 0, 1, sA, rA, 0, p1)
        a1.start()

        pB = compute_partial(1)
        cB[0] = pB.astype(jnp.bfloat16)
        b1 = exchange(cB, 0, 1, sB, rB, 0, p2)
        b1.start()

        a1.wait()
        pairA = pA + cA[1].astype(jnp.float32)
        cA[2] = pairA.astype(jnp.bfloat16)
        a2 = exchange(cA, 2, 3, sA, rA, 1, p2)
        a2.start()

        b1.wait()
        pairB = pB + cB[1].astype(jnp.float32)
        cB[2] = pairB.astype(jnp.bfloat16)
        b2 = exchange(cB, 2, 3, sB, rB, 1, p1)
        b2.start()

        a2.wait()
        out_ref[pl.ds(0, SQ), :] = pairA + cA[3].astype(jnp.float32)
        b2.wait()
        out_ref[pl.ds(SQ, SQ), :] = pairB + cB[3].astype(jnp.float32)

    return pl.pallas_call(
        body,
        out_shape=jax.ShapeDtypeStruct((B * SQ, D), jnp.float32),
        in_specs=[pl.BlockSpec(memory_space=pltpu.VMEM)] * 5,
        out_specs=pl.BlockSpec(memory_space=pltpu.VMEM),
        scratch_shapes=[
            pltpu.VMEM((4, SQ, D), jnp.bfloat16),
            pltpu.VMEM((4, SQ, D), jnp.bfloat16),
            pltpu.VMEM((SQ, Hq * DH), jnp.bfloat16),
            pltpu.SemaphoreType.DMA((2,)),
            pltpu.SemaphoreType.DMA((2,)),
            pltpu.SemaphoreType.DMA((2,)),
            pltpu.SemaphoreType.DMA((2,)),
        ],
        compiler_params=pltpu.CompilerParams(collective_id=0),
    )(xb, Wq, Wo, K, V)


def kernel(x, Wq, Wo, K_ext, V_ext):
    my = lax.axis_index("i")
    Hq = Wq.shape[1] // DH

    xb = x.astype(jnp.bfloat16)
    K = lax.dynamic_slice_in_dim(K_ext, my * Hq, Hq, axis=2)
    V = lax.dynamic_slice_in_dim(V_ext, my * Hq, Hq, axis=2)
    K = jnp.transpose(K, (0, 2, 1, 3)).astype(jnp.bfloat16)
    V = jnp.transpose(V, (0, 2, 1, 3)).astype(jnp.bfloat16)

    out = _fused(
        xb, Wq.astype(jnp.bfloat16), Wo.astype(jnp.bfloat16), K, V
    )
    return out.reshape(B, SQ, D)


# device time: 30122 ns/iter; 1.0358x vs baseline; 1.0358x over previous
import jax
import jax.numpy as jnp
from jax import lax
from jax.experimental import pallas as pl
from jax.experimental.pallas import tpu as pltpu

N_DEV = 4
DH = 64
B = 2
SQ = 256
D = 768


def _fused(xb, Wq, Wo, K, V):
    Hq = K.shape[1]

    def body(x_ref, wq_ref, wo_ref, k_ref, v_ref, out_ref,
             cA, cB, o_buf, sA, rA, sB, rB):
        my = lax.axis_index("i")
        p1 = my ^ 1
        p2 = 3 - my

        barrier_sem = pltpu.get_barrier_semaphore()
        for nbr in (p1, p2):
            pl.semaphore_signal(
                barrier_sem, inc=1,
                device_id=(nbr,), device_id_type=pl.DeviceIdType.MESH,
            )
        pl.semaphore_wait(barrier_sem, 2)

        def compute_partial(b):
            q = (jnp.dot(
                x_ref[b], wq_ref[...], preferred_element_type=jnp.float32
            ) * 0.125).astype(jnp.bfloat16)
            for h in range(Hq):
                qh = q[:, DH * h:DH * (h + 1)]
                s = jnp.dot(
                    qh, k_ref[b, h],
                    preferred_element_type=jnp.float32,
                )
                m = jnp.max(s, axis=1, keepdims=True)
                p = jnp.exp(s - m)
                l = jnp.sum(p, axis=1, keepdims=True)
                oh = jnp.dot(
                    p.astype(jnp.bfloat16), v_ref[b, h],
                    preferred_element_type=jnp.float32,
                )
                o_buf[:, DH * h:DH * (h + 1)] = (oh / l).astype(jnp.bfloat16)
            return jnp.dot(
                o_buf[...], wo_ref[...], preferred_element_type=jnp.float32
            )

        def exchange(c, slot_src, slot_dst, sem_s, sem_r, idx, tgt):
            return pltpu.make_async_remote_copy(
                src_ref=c.at[slot_src], dst_ref=c.at[slot_dst],
                send_sem=sem_s.at[idx], recv_sem=sem_r.at[idx],
                device_id=(tgt,), device_id_type=pl.DeviceIdType.MESH,
            )

        pA = compute_partial(0)
        cA[0] = pA.astype(jnp.bfloat16)
        a1 = exchange(cA, 0, 1, sA, rA, 0, p1)
        a1.start()

        pB = compute_partial(1)
        cB[0] = pB.astype(jnp.bfloat16)
        b1 = exchange(cB, 0, 1, sB, rB, 0, p2)
        b1.start()

        a1.wait()
        pairA = pA + cA[1].astype(jnp.float32)
        cA[2] = pairA.astype(jnp.bfloat16)
        a2 = exchange(cA, 2, 3, sA, rA, 1, p2)
        a2.start()

        b1.wait()
        pairB = pB + cB[1].astype(jnp.float32)
        cB[2] = pairB.astype(jnp.bfloat16)
        b2 = exchange(cB, 2, 3, sB, rB, 1, p1)
        b2.start()

        a2.wait()
        out_ref[pl.ds(0, SQ), :] = pairA + cA[3].astype(jnp.float32)
        b2.wait()
        out_ref[pl.ds(SQ, SQ), :] = pairB + cB[3].astype(jnp.float32)

    return pl.pallas_call(
        body,
        out_shape=jax.ShapeDtypeStruct((B * SQ, D), jnp.float32),
        in_specs=[pl.BlockSpec(memory_space=pltpu.VMEM)] * 5,
        out_specs=pl.BlockSpec(memory_space=pltpu.VMEM),
        scratch_shapes=[
            pltpu.VMEM((4, SQ, D), jnp.bfloat16),
            pltpu.VMEM((4, SQ, D), jnp.bfloat16),
            pltpu.VMEM((SQ, Hq * DH), jnp.bfloat16),
            pltpu.SemaphoreType.DMA((2,)),
            pltpu.SemaphoreType.DMA((2,)),
            pltpu.SemaphoreType.DMA((2,)),
            pltpu.SemaphoreType.DMA((2,)),
        ],
        compiler_params=pltpu.CompilerParams(collective_id=0),
    )(xb, Wq, Wo, K, V)


def kernel(x, Wq, Wo, K_ext, V_ext):
    my = lax.axis_index("i")
    Hq = Wq.shape[1] // DH

    xb = x.astype(jnp.bfloat16)
    K = lax.dynamic_slice_in_dim(K_ext, my * Hq, Hq, axis=2)
    V = lax.dynamic_slice_in_dim(V_ext, my * Hq, Hq, axis=2)
    K = jnp.transpose(K, (0, 2, 3, 1)).astype(jnp.bfloat16)
    V = jnp.transpose(V, (0, 2, 1, 3)).astype(jnp.bfloat16)

    out = _fused(
        xb, Wq.astype(jnp.bfloat16), Wo.astype(jnp.bfloat16), K, V
    )
    return out.reshape(B, SQ, D)


# device time: 25555 ns/iter; 1.2209x vs baseline; 1.1787x over previous
import jax
import jax.numpy as jnp
from jax import lax
from jax.experimental import pallas as pl
from jax.experimental.pallas import tpu as pltpu

N_DEV = 4
DH = 64
B = 2
SQ = 256
D = 768
HALF = D // 2

CHUNKS = [(0, 0, 256, 0), (1, 0, 192, 256), (1, 192, 64, 448)]


def _fused(xb, Wq, Wo, K, V):
    Hq = K.shape[1]

    def body(x_ref, wq_ref, wo_ref, k_ref, v_ref, out_ref,
             s1, s2, r1L, r1R, r2L, r2R, o_buf,
             sph1, rph1, sph2, rph2):
        my = lax.axis_index("i")
        p1 = my ^ 1
        p2 = 3 - my

        barrier_sem = pltpu.get_barrier_semaphore()
        for nbr in (p1, p2):
            pl.semaphore_signal(
                barrier_sem, inc=1,
                device_id=(nbr,), device_id_type=pl.DeviceIdType.MESH,
            )
        pl.semaphore_wait(barrier_sem, 2)

        def rdma(src, dst, ss, rs, tgt):
            return pltpu.make_async_remote_copy(
                src_ref=src, dst_ref=dst, send_sem=ss, recv_sem=rs,
                device_id=(tgt,), device_id_type=pl.DeviceIdType.MESH,
            )

        def compute_chunk(c):
            b, q0, nr, _ = CHUNKS[c]
            q = (jnp.dot(
                x_ref[b, pl.ds(q0, nr), :], wq_ref[...],
                preferred_element_type=jnp.float32,
            ) * 0.125).astype(jnp.bfloat16)
            for h in range(Hq):
                qh = q[:, DH * h:DH * (h + 1)]
                s = lax.dot_general(
                    qh, k_ref[b, h],
                    (((1,), (1,)), ((), ())),
                    preferred_element_type=jnp.float32,
                )
                p = jnp.exp(s).astype(jnp.bfloat16)
                ov = jnp.dot(p, v_ref[b, h],
                             preferred_element_type=jnp.float32)
                o_buf[pl.ds(0, nr), pl.ds(DH * h, DH)] = (
                    ov[:, 0:DH] * (1.0 / ov[:, DH:DH + 1])
                ).astype(jnp.bfloat16)
            return jnp.dot(o_buf[pl.ds(0, nr), :], wo_ref[...],
                           preferred_element_type=jnp.float32)

        def start_ph1(c):
            b, q0, nr, r0 = CHUNKS[c]
            part = compute_chunk(c)
            s1[pl.ds(r0, nr), :] = part.astype(jnp.bfloat16)
            dL = rdma(s1.at[pl.ds(r0, nr), pl.ds(0, HALF)],
                      r1L.at[pl.ds(r0, nr), :],
                      sph1.at[2 * c], rph1.at[2 * c], p1)
            dR = rdma(s1.at[pl.ds(r0, nr), pl.ds(HALF, HALF)],
                      r1R.at[pl.ds(r0, nr), :],
                      sph1.at[2 * c + 1], rph1.at[2 * c + 1], p2)
            dL.start()
            dR.start()
            return part, dL, dR

        def start_ph2(c, st):
            part, dL, dR = st
            _, _, nr, r0 = CHUNKS[c]
            dL.wait()
            dR.wait()
            pairL = part[:, 0:HALF] + r1L[pl.ds(r0, nr), :].astype(jnp.float32)
            pairR = part[:, HALF:D] + r1R[pl.ds(r0, nr), :].astype(jnp.float32)
            s2[pl.ds(r0, nr), pl.ds(0, HALF)] = pairL.astype(jnp.bfloat16)
            s2[pl.ds(r0, nr), pl.ds(HALF, HALF)] = pairR.astype(jnp.bfloat16)
            eL = rdma(s2.at[pl.ds(r0, nr), pl.ds(0, HALF)],
                      r2L.at[pl.ds(r0, nr), :],
                      sph2.at[2 * c], rph2.at[2 * c], p2)
            eR = rdma(s2.at[pl.ds(r0, nr), pl.ds(HALF, HALF)],
                      r2R.at[pl.ds(r0, nr), :],
                      sph2.at[2 * c + 1], rph2.at[2 * c + 1], p1)
            eL.start()
            eR.start()
            return pairL, pairR, eL, eR

        def finish(c, st):
            pairL, pairR, eL, eR = st
            _, _, nr, r0 = CHUNKS[c]
            eL.wait()
            eR.wait()
            out_ref[pl.ds(r0, nr), pl.ds(0, HALF)] = (
                pairL + r2L[pl.ds(r0, nr), :].astype(jnp.float32))
            out_ref[pl.ds(r0, nr), pl.ds(HALF, HALF)] = (
                pairR + r2R[pl.ds(r0, nr), :].astype(jnp.float32))

        st0 = start_ph1(0)
        st1 = start_ph1(1)
        f0 = start_ph2(0, st0)
        st2 = start_ph1(2)
        f1 = start_ph2(1, st1)
        finish(0, f0)
        f2 = start_ph2(2, st2)
        finish(1, f1)
        finish(2, f2)

    return pl.pallas_call(
        body,
        out_shape=jax.ShapeDtypeStruct((B * SQ, D), jnp.float32),
        in_specs=[pl.BlockSpec(memory_space=pltpu.VMEM)] * 5,
        out_specs=pl.BlockSpec(memory_space=pltpu.VMEM),
        scratch_shapes=[
            pltpu.VMEM((B * SQ, D), jnp.bfloat16),
            pltpu.VMEM((B * SQ, D), jnp.bfloat16),
            pltpu.VMEM((B * SQ, HALF), jnp.bfloat16),
            pltpu.VMEM((B * SQ, HALF), jnp.bfloat16),
            pltpu.VMEM((B * SQ, HALF), jnp.bfloat16),
            pltpu.VMEM((B * SQ, HALF), jnp.bfloat16),
            pltpu.VMEM((SQ, 8 * DH), jnp.bfloat16),
            pltpu.SemaphoreType.DMA((6,)),
            pltpu.SemaphoreType.DMA((6,)),
            pltpu.SemaphoreType.DMA((6,)),
            pltpu.SemaphoreType.DMA((6,)),
        ],
        compiler_params=pltpu.CompilerParams(collective_id=0),
    )(xb, Wq, Wo, K, V)


def kernel(x, Wq, Wo, K_ext, V_ext):
    my = lax.axis_index("i")
    Hq = Wq.shape[1] // DH

    xb = x.astype(jnp.bfloat16)
    K = lax.dynamic_slice_in_dim(K_ext, my * Hq, Hq, axis=2)
    V = lax.dynamic_slice_in_dim(V_ext, my * Hq, Hq, axis=2)
    K = jnp.transpose(K, (0, 2, 1, 3)).astype(jnp.bfloat16)
    V = jnp.transpose(V, (0, 2, 1, 3)).astype(jnp.bfloat16)
    V = jnp.concatenate(
        [V,
         jnp.ones((B, Hq, 512, 1), jnp.bfloat16),
         jnp.zeros((B, Hq, 512, DH - 1), jnp.bfloat16)],
        axis=3,
    )

    out = _fused(
        xb, Wq.astype(jnp.bfloat16), Wo.astype(jnp.bfloat16), K, V
    )
    return out.reshape(B, SQ, D)


# device time: 25461 ns/iter; 1.2254x vs baseline; 1.0037x over previous
import jax
import jax.numpy as jnp
from jax import lax
from jax.experimental import pallas as pl
from jax.experimental.pallas import tpu as pltpu

N_DEV = 4
DH = 64
B = 2
SQ = 256
D = 768
HALF = D // 2

CHUNKS = [(0, 0, 256, 0), (1, 0, 192, 256), (1, 192, 64, 448)]


def _fused(xb, Wq, Wo, K, V):
    Hq = K.shape[1]

    def body(x_ref, wq_ref, wo_ref, k_ref, v_ref, out_ref,
             s1, s2, r1L, r1R, r2L, r2R, o_buf,
             sph1, rph1, sph2, rph2):
        my = lax.axis_index("i")
        p1 = my ^ 1
        p2 = 3 - my

        barrier_sem = pltpu.get_barrier_semaphore()
        for nbr in (p1, p2):
            pl.semaphore_signal(
                barrier_sem, inc=1,
                device_id=(nbr,), device_id_type=pl.DeviceIdType.MESH,
            )
        pl.semaphore_wait(barrier_sem, 2)

        def rdma(src, dst, ss, rs, tgt):
            return pltpu.make_async_remote_copy(
                src_ref=src, dst_ref=dst, send_sem=ss, recv_sem=rs,
                device_id=(tgt,), device_id_type=pl.DeviceIdType.MESH,
            )

        def compute_chunk(c):
            b, q0, nr, _ = CHUNKS[c]
            q = (jnp.dot(
                x_ref[b, pl.ds(q0, nr), :], wq_ref[...],
                preferred_element_type=jnp.float32,
            ) * 0.125).astype(jnp.bfloat16)
            for h in range(Hq):
                qh = q[:, DH * h:DH * (h + 1)]
                s = lax.dot_general(
                    qh, k_ref[b, h],
                    (((1,), (1,)), ((), ())),
                    preferred_element_type=jnp.float32,
                )
                p = jnp.exp(s).astype(jnp.bfloat16)
                ov = jnp.dot(p, v_ref[b, h],
                             preferred_element_type=jnp.float32)
                o_buf[pl.ds(0, nr), pl.ds(DH * h, DH)] = (
                    ov[:, 0:DH] * (1.0 / ov[:, DH:DH + 1])
                ).astype(jnp.bfloat16)
            return jnp.dot(o_buf[pl.ds(0, nr), :], wo_ref[...],
                           preferred_element_type=jnp.float32)

        def start_ph1(c):
            b, q0, nr, r0 = CHUNKS[c]
            part = compute_chunk(c)
            s1[pl.ds(r0, nr), :] = part.astype(jnp.bfloat16)
            dL = rdma(s1.at[pl.ds(r0, nr), pl.ds(0, HALF)],
                      r1L.at[pl.ds(r0, nr), :],
                      sph1.at[2 * c], rph1.at[2 * c], p1)
            dR = rdma(s1.at[pl.ds(r0, nr), pl.ds(HALF, HALF)],
                      r1R.at[pl.ds(r0, nr), :],
                      sph1.at[2 * c + 1], rph1.at[2 * c + 1], p2)
            dL.start()
            dR.start()
            return part, dL, dR

        def start_ph2(c, st):
            part, dL, dR = st
            _, _, nr, r0 = CHUNKS[c]
            dL.wait()
            dR.wait()
            pairL = part[:, 0:HALF] + r1L[pl.ds(r0, nr), :].astype(jnp.float32)
            pairR = part[:, HALF:D] + r1R[pl.ds(r0, nr), :].astype(jnp.float32)
            s2[pl.ds(r0, nr), pl.ds(0, HALF)] = pairL.astype(jnp.bfloat16)
            s2[pl.ds(r0, nr), pl.ds(HALF, HALF)] = pairR.astype(jnp.bfloat16)
            eL = rdma(s2.at[pl.ds(r0, nr), pl.ds(0, HALF)],
                      r2L.at[pl.ds(r0, nr), :],
                      sph2.at[2 * c], rph2.at[2 * c], p2)
            eR = rdma(s2.at[pl.ds(r0, nr), pl.ds(HALF, HALF)],
                      r2R.at[pl.ds(r0, nr), :],
                      sph2.at[2 * c + 1], rph2.at[2 * c + 1], p1)
            eL.start()
            eR.start()
            return pairL, pairR, eL, eR

        def finish(c, st):
            pairL, pairR, eL, eR = st
            _, _, nr, r0 = CHUNKS[c]
            eL.wait()
            eR.wait()
            out_ref[pl.ds(r0, nr), pl.ds(0, HALF)] = (
                pairL + r2L[pl.ds(r0, nr), :].astype(jnp.float32))
            out_ref[pl.ds(r0, nr), pl.ds(HALF, HALF)] = (
                pairR + r2R[pl.ds(r0, nr), :].astype(jnp.float32))

        st0 = start_ph1(0)
        st1 = start_ph1(1)
        f0 = start_ph2(0, st0)
        st2 = start_ph1(2)
        f1 = start_ph2(1, st1)
        finish(0, f0)
        f2 = start_ph2(2, st2)
        finish(1, f1)
        finish(2, f2)

    return pl.pallas_call(
        body,
        out_shape=jax.ShapeDtypeStruct((B * SQ, D), jnp.float32),
        in_specs=[pl.BlockSpec(memory_space=pltpu.VMEM)] * 5,
        out_specs=pl.BlockSpec(memory_space=pltpu.VMEM),
        scratch_shapes=[
            pltpu.VMEM((B * SQ, D), jnp.bfloat16),
            pltpu.VMEM((B * SQ, D), jnp.bfloat16),
            pltpu.VMEM((B * SQ, HALF), jnp.bfloat16),
            pltpu.VMEM((B * SQ, HALF), jnp.bfloat16),
            pltpu.VMEM((B * SQ, HALF), jnp.bfloat16),
            pltpu.VMEM((B * SQ, HALF), jnp.bfloat16),
            pltpu.VMEM((SQ, 8 * DH), jnp.bfloat16),
            pltpu.SemaphoreType.DMA((6,)),
            pltpu.SemaphoreType.DMA((6,)),
            pltpu.SemaphoreType.DMA((6,)),
            pltpu.SemaphoreType.DMA((6,)),
        ],
        compiler_params=pltpu.CompilerParams(collective_id=0),
    )(xb, Wq, Wo, K, V)


def kernel(x, Wq, Wo, K_ext, V_ext):
    my = lax.axis_index("i")
    Hq = Wq.shape[1] // DH

    xb = x.astype(jnp.bfloat16)
    K = lax.dynamic_slice_in_dim(K_ext, my * Hq, Hq, axis=2)
    V = lax.dynamic_slice_in_dim(V_ext, my * Hq, Hq, axis=2)
    K = jnp.transpose(K.astype(jnp.bfloat16), (0, 2, 1, 3))
    V = jnp.transpose(V.astype(jnp.bfloat16), (0, 2, 1, 3))
    V = jnp.concatenate(
        [V, jnp.ones((B, Hq, 512, 1), jnp.bfloat16)], axis=3
    )

    out = _fused(
        xb, Wq.astype(jnp.bfloat16), Wo.astype(jnp.bfloat16), K, V
    )
    return out.reshape(B, SQ, D)


# device time: 24780 ns/iter; 1.2591x vs baseline; 1.0275x over previous
import jax
import jax.numpy as jnp
from jax import lax
from jax.experimental import pallas as pl
from jax.experimental.pallas import tpu as pltpu

N_DEV = 4
DH = 64
B = 2
SQ = 256
D = 768
HALF = D // 2

CHUNKS = [(0, 0, 256, 0), (1, 0, 192, 256), (1, 192, 64, 448)]


def _fused(xb, Wq, Wo, K, V):
    Hq = K.shape[2] // DH

    def body(x_ref, wq_ref, wo_ref, k_ref, v_ref, out_ref,
             s1, s2, r1L, r1R, r2L, r2R, o_buf,
             sph1, rph1, sph2, rph2):
        my = lax.axis_index("i")
        p1 = my ^ 1
        p2 = 3 - my

        barrier_sem = pltpu.get_barrier_semaphore()
        for nbr in (p1, p2):
            pl.semaphore_signal(
                barrier_sem, inc=1,
                device_id=(nbr,), device_id_type=pl.DeviceIdType.MESH,
            )
        pl.semaphore_wait(barrier_sem, 2)

        def rdma(src, dst, ss, rs, tgt):
            return pltpu.make_async_remote_copy(
                src_ref=src, dst_ref=dst, send_sem=ss, recv_sem=rs,
                device_id=(tgt,), device_id_type=pl.DeviceIdType.MESH,
            )

        def compute_chunk(c):
            b, q0, nr, _ = CHUNKS[c]
            q = (jnp.dot(
                x_ref[b, pl.ds(q0, nr), :], wq_ref[...],
                preferred_element_type=jnp.float32,
            ) * 0.125).astype(jnp.bfloat16)
            for h in range(Hq):
                qh = q[:, DH * h:DH * (h + 1)]
                s = lax.dot_general(
                    qh, k_ref[b, :, pl.ds(DH * h, DH)],
                    (((1,), (1,)), ((), ())),
                    preferred_element_type=jnp.float32,
                )
                p32 = jnp.exp(s)
                rl = 1.0 / jnp.sum(p32, axis=1, keepdims=True)
                ov = jnp.dot(p32.astype(jnp.bfloat16),
                             v_ref[b, :, pl.ds(DH * h, DH)],
                             preferred_element_type=jnp.float32)
                o_buf[pl.ds(0, nr), pl.ds(DH * h, DH)] = (
                    ov * rl
                ).astype(jnp.bfloat16)
            return jnp.dot(o_buf[pl.ds(0, nr), :], wo_ref[...],
                           preferred_element_type=jnp.float32)

        def start_ph1(c):
            b, q0, nr, r0 = CHUNKS[c]
            part = compute_chunk(c)
            s1[pl.ds(r0, nr), :] = part.astype(jnp.bfloat16)
            dL = rdma(s1.at[pl.ds(r0, nr), pl.ds(0, HALF)],
                      r1L.at[pl.ds(r0, nr), :],
                      sph1.at[2 * c], rph1.at[2 * c], p1)
            dR = rdma(s1.at[pl.ds(r0, nr), pl.ds(HALF, HALF)],
                      r1R.at[pl.ds(r0, nr), :],
                      sph1.at[2 * c + 1], rph1.at[2 * c + 1], p2)
            dL.start()
            dR.start()
            return part, dL, dR

        def start_ph2(c, st):
            part, dL, dR = st
            _, _, nr, r0 = CHUNKS[c]
            dL.wait()
            dR.wait()
            pairL = part[:, 0:HALF] + r1L[pl.ds(r0, nr), :].astype(jnp.float32)
            pairR = part[:, HALF:D] + r1R[pl.ds(r0, nr), :].astype(jnp.float32)
            s2[pl.ds(r0, nr), pl.ds(0, HALF)] = pairL.astype(jnp.bfloat16)
            s2[pl.ds(r0, nr), pl.ds(HALF, HALF)] = pairR.astype(jnp.bfloat16)
            eL = rdma(s2.at[pl.ds(r0, nr), pl.ds(0, HALF)],
                      r2L.at[pl.ds(r0, nr), :],
                      sph2.at[2 * c], rph2.at[2 * c], p2)
            eR = rdma(s2.at[pl.ds(r0, nr), pl.ds(HALF, HALF)],
                      r2R.at[pl.ds(r0, nr), :],
                      sph2.at[2 * c + 1], rph2.at[2 * c + 1], p1)
            eL.start()
            eR.start()
            return pairL, pairR, eL, eR

        def finish(c, st):
            pairL, pairR, eL, eR = st
            _, _, nr, r0 = CHUNKS[c]
            eL.wait()
            eR.wait()
            out_ref[pl.ds(r0, nr), pl.ds(0, HALF)] = (
                pairL + r2L[pl.ds(r0, nr), :].astype(jnp.float32))
            out_ref[pl.ds(r0, nr), pl.ds(HALF, HALF)] = (
                pairR + r2R[pl.ds(r0, nr), :].astype(jnp.float32))

        st0 = start_ph1(0)
        st1 = start_ph1(1)
        f0 = start_ph2(0, st0)
        st2 = start_ph1(2)
        f1 = start_ph2(1, st1)
        finish(0, f0)
        f2 = start_ph2(2, st2)
        finish(1, f1)
        finish(2, f2)

    return pl.pallas_call(
        body,
        out_shape=jax.ShapeDtypeStruct((B * SQ, D), jnp.float32),
        in_specs=[pl.BlockSpec(memory_space=pltpu.VMEM)] * 5,
        out_specs=pl.BlockSpec(memory_space=pltpu.VMEM),
        scratch_shapes=[
            pltpu.VMEM((B * SQ, D), jnp.bfloat16),
            pltpu.VMEM((B * SQ, D), jnp.bfloat16),
            pltpu.VMEM((B * SQ, HALF), jnp.bfloat16),
            pltpu.VMEM((B * SQ, HALF), jnp.bfloat16),
            pltpu.VMEM((B * SQ, HALF), jnp.bfloat16),
            pltpu.VMEM((B * SQ, HALF), jnp.bfloat16),
            pltpu.VMEM((SQ, 8 * DH), jnp.bfloat16),
            pltpu.SemaphoreType.DMA((6,)),
            pltpu.SemaphoreType.DMA((6,)),
            pltpu.SemaphoreType.DMA((6,)),
            pltpu.SemaphoreType.DMA((6,)),
        ],
        compiler_params=pltpu.CompilerParams(collective_id=0),
    )(xb, Wq, Wo, K, V)


def kernel(x, Wq, Wo, K_ext, V_ext):
    my = lax.axis_index("i")
    Hq = Wq.shape[1] // DH

    xb = x.astype(jnp.bfloat16)
    Skv = K_ext.shape[1]
    K = lax.dynamic_slice_in_dim(K_ext, my * Hq, Hq, axis=2)
    V = lax.dynamic_slice_in_dim(V_ext, my * Hq, Hq, axis=2)
    K = K.astype(jnp.bfloat16).reshape(B, Skv, Hq * DH)
    V = V.astype(jnp.bfloat16).reshape(B, Skv, Hq * DH)

    out = _fused(
        xb, Wq.astype(jnp.bfloat16), Wo.astype(jnp.bfloat16), K, V
    )
    return out.reshape(B, SQ, D)
